# Initial kernel scaffold; baseline (speedup 1.0000x reference)
import jax
import jax.numpy as jnp
from jax.experimental import pallas as pl


def kernel(rule_count, rule_emb, candidate_set, all_r, relation_table, ln_gamma, ln_beta, W1, b1, W2, b2, bias):
    raise NotImplementedError("write your pallas kernel here")


# speedup vs baseline: 6.3676x; 6.3676x over previous
import functools

import jax
import jax.numpy as jnp
from jax import lax
from jax.experimental import pallas as pl
from jax.experimental.pallas import tpu as pltpu
from jax.experimental.pallas import tpu_sc as plsc

R, C, H, B, E = 64, 200000, 16, 16, 50000
BE = B * E
BLK = 4096
GRID = (C + BLK - 1) // BLK
CPAD = GRID * BLK
NW = 32
CHUNK = BE // NW
TILE = 512
GROUPS = TILE // 16


def _dense_body(rc_ref, emb_ref, rel_ref, gam_ref, bet_ref, w1_ref, b1_ref,
                w2_ref, b2_ref, s_ref):
    rc = rc_ref[...]
    msg = lax.dot_general(emb_ref[...], rc, (((0,), (0,)), ((), ())),
                          preferred_element_type=jnp.float32)
    deg = jnp.sum(rc, axis=0, keepdims=True) + 1e-6
    x = msg / deg
    mu = jnp.mean(x, axis=0, keepdims=True)
    var = jnp.mean((x - mu) * (x - mu), axis=0, keepdims=True)
    x = (x - mu) * lax.rsqrt(var + 1e-5) * gam_ref[...] + bet_ref[...]
    x = jnp.maximum(x, 0.0)
    feat = jnp.concatenate([x, jnp.broadcast_to(rel_ref[...], x.shape)],
                           axis=0)
    h = lax.dot_general(w1_ref[...], feat, (((0,), (0,)), ((), ())),
                        preferred_element_type=jnp.float32)
    h = jnp.maximum(h + b1_ref[...], 0.0)
    s = lax.dot_general(w2_ref[...], h, (((0,), (0,)), ((), ())),
                        preferred_element_type=jnp.float32)
    s_ref[...] = s + b2_ref[...]


def _dense_scores(rule_count, rule_emb, rel, ln_gamma, ln_beta, W1, b1, W2, b2):
    return pl.pallas_call(
        _dense_body,
        grid=(GRID,),
        in_specs=[
            pl.BlockSpec((R, BLK), lambda i: (0, i)),
            pl.BlockSpec((R, H), lambda i: (0, 0)),
            pl.BlockSpec((H, 1), lambda i: (0, 0)),
            pl.BlockSpec((H, 1), lambda i: (0, 0)),
            pl.BlockSpec((H, 1), lambda i: (0, 0)),
            pl.BlockSpec((2 * H, 128), lambda i: (0, 0)),
            pl.BlockSpec((128, 1), lambda i: (0, 0)),
            pl.BlockSpec((128, 1), lambda i: (0, 0)),
            pl.BlockSpec((1, 1), lambda i: (0, 0)),
        ],
        out_specs=pl.BlockSpec((1, BLK), lambda i: (0, i)),
        out_shape=jax.ShapeDtypeStruct((1, CPAD), jnp.float32),
        compiler_params=pltpu.CompilerParams(
            dimension_semantics=("arbitrary",)),
    )(rule_count, rule_emb, rel, ln_gamma, ln_beta, W1, b1, W2, b2)


def _sc_scatter(cand_pad, s_pad, bias, lo_arr, hi_arr):
    mesh = plsc.VectorSubcoreMesh(core_axis_name="c", subcore_axis_name="s")

    @functools.partial(
        pl.kernel,
        mesh=mesh,
        out_type=jax.ShapeDtypeStruct((BE,), jnp.float32),
        compiler_params=pltpu.CompilerParams(needs_layout_passes=False),
        scratch_types=[
            pltpu.VMEM((CHUNK,), jnp.float32),
            pltpu.VMEM((TILE + 16,), jnp.int32),
            pltpu.VMEM((TILE,), jnp.float32),
            pltpu.VMEM((NW + 16,), jnp.int32),
            pltpu.VMEM((NW + 16,), jnp.int32),
        ],
    )
    def k(cand_hbm, s_hbm, bias_hbm, lo_hbm, hi_hbm, out_hbm,
          chunk, cbuf, sbuf, lov, hiv):
        cid = lax.axis_index("c")
        sid = lax.axis_index("s")
        wid = cid * 16 + sid
        ostart = pl.multiple_of(wid * CHUNK, 8)
        oend = ostart + CHUNK
        estart = pl.multiple_of(lax.rem(ostart, E), 8)
        pltpu.sync_copy(bias_hbm.at[pl.ds(estart, CHUNK)], chunk)
        pltpu.sync_copy(lo_hbm, lov)
        pltpu.sync_copy(hi_hbm, hiv)
        lo = lov[pl.ds(wid, 16)][0]
        hi = hiv[pl.ds(wid, 16)][0]
        lo8 = pl.multiple_of(lo - lax.rem(lo, 8), 8)
        ntiles = (hi - lo8 + (TILE - 1)) // TILE

        def tile_body(t, carry):
            base = pl.multiple_of(lo8 + t * TILE, 8)
            pltpu.sync_copy(cand_hbm.at[pl.ds(base, TILE + 16)], cbuf)
            pltpu.sync_copy(s_hbm.at[pl.ds(base, TILE)], sbuf)
            for g in range(GROUPS):
                off = g * 16
                vc = cbuf[pl.ds(off, 16)]
                vn = cbuf[pl.ds(off + 1, 16)]
                vs = sbuf[pl.ds(off, 16)]
                idx = vc - ostart
                keep = (vc >= ostart) & (vc < oend) & (vc != vn)
                bv = plsc.load_gather(chunk, [idx], mask=keep)
                plsc.store_scatter(chunk, [idx], vs + bv, mask=keep)
            return carry

        lax.fori_loop(0, ntiles, tile_body, 0)
        pltpu.sync_copy(chunk, out_hbm.at[pl.ds(ostart, CHUNK)])

    return k(cand_pad, s_pad, bias, lo_arr, hi_arr)


def kernel(rule_count, rule_emb, candidate_set, all_r, relation_table,
           ln_gamma, ln_beta, W1, b1, W2, b2, bias):
    rel = relation_table[all_r].reshape(H, 1)
    s2d = _dense_scores(rule_count, rule_emb, rel,
                        ln_gamma.reshape(H, 1), ln_beta.reshape(H, 1),
                        W1, b1.reshape(128, 1), W2, b2.reshape(1, 1))
    s_pad = s2d.reshape(CPAD)
    cand_pad = jnp.concatenate(
        [candidate_set, jnp.full((CPAD - C,), BE, dtype=jnp.int32)])
    edges = jnp.arange(NW + 1, dtype=jnp.int32) * CHUNK
    bounds = jnp.searchsorted(candidate_set, edges, side="left").astype(jnp.int32)
    zpad = jnp.zeros((15,), jnp.int32)
    lo_arr = jnp.concatenate([bounds[:NW], zpad, zpad[:1]])
    hi_arr = jnp.concatenate([bounds[1:], zpad, zpad[:1]])
    out_flat = _sc_scatter(cand_pad, s_pad, bias, lo_arr, hi_arr)
    return out_flat.reshape(B, E)


# speedup vs baseline: 10.6455x; 1.6718x over previous
import functools

import jax
import jax.numpy as jnp
from jax import lax
from jax.experimental import pallas as pl
from jax.experimental.pallas import tpu as pltpu
from jax.experimental.pallas import tpu_sc as plsc

R, C, H, B, E = 64, 200000, 16, 16, 50000
BE = B * E
BLK = 8192
GRID = (C + BLK - 1) // BLK
CPAD = GRID * BLK
NW = 32
CHUNK = BE // NW
TILE = 2048
GROUPS = TILE // 16


def _dense_body(rc_ref, emb_ref, rel_ref, gam_ref, bet_ref, w1_ref, b1_ref,
                w2_ref, b2_ref, s_ref):
    rc = rc_ref[...]
    msg = lax.dot_general(emb_ref[...], rc, (((0,), (0,)), ((), ())),
                          preferred_element_type=jnp.float32)
    deg = jnp.sum(rc, axis=0, keepdims=True) + 1e-6
    x = msg / deg
    mu = jnp.mean(x, axis=0, keepdims=True)
    var = jnp.mean((x - mu) * (x - mu), axis=0, keepdims=True)
    x = (x - mu) * lax.rsqrt(var + 1e-5) * gam_ref[...] + bet_ref[...]
    x = jnp.maximum(x, 0.0)
    feat = jnp.concatenate([x, jnp.broadcast_to(rel_ref[...], x.shape)],
                           axis=0)
    h = lax.dot_general(w1_ref[...], feat, (((0,), (0,)), ((), ())),
                        preferred_element_type=jnp.float32)
    h = jnp.maximum(h + b1_ref[...], 0.0)
    s = lax.dot_general(w2_ref[...], h, (((0,), (0,)), ((), ())),
                        preferred_element_type=jnp.float32)
    s_ref[...] = s + b2_ref[...]


def _dense_scores(rule_count, rule_emb, rel, ln_gamma, ln_beta, W1, b1, W2, b2):
    return pl.pallas_call(
        _dense_body,
        grid=(GRID,),
        in_specs=[
            pl.BlockSpec((R, BLK), lambda i: (0, i)),
            pl.BlockSpec((R, H), lambda i: (0, 0)),
            pl.BlockSpec((H, 1), lambda i: (0, 0)),
            pl.BlockSpec((H, 1), lambda i: (0, 0)),
            pl.BlockSpec((H, 1), lambda i: (0, 0)),
            pl.BlockSpec((2 * H, 128), lambda i: (0, 0)),
            pl.BlockSpec((128, 1), lambda i: (0, 0)),
            pl.BlockSpec((128, 1), lambda i: (0, 0)),
            pl.BlockSpec((1, 1), lambda i: (0, 0)),
        ],
        out_specs=pl.BlockSpec((1, BLK), lambda i: (0, i)),
        out_shape=jax.ShapeDtypeStruct((1, CPAD), jnp.float32),
        compiler_params=pltpu.CompilerParams(
            dimension_semantics=("arbitrary",)),
    )(rule_count, rule_emb, rel, ln_gamma, ln_beta, W1, b1, W2, b2)


def _sc_scatter(cand_pad, s_pad, bias, lo_arr, hi_arr):
    mesh = plsc.VectorSubcoreMesh(core_axis_name="c", subcore_axis_name="s")

    @functools.partial(
        pl.kernel,
        mesh=mesh,
        out_type=jax.ShapeDtypeStruct((BE,), jnp.float32),
        compiler_params=pltpu.CompilerParams(needs_layout_passes=False),
        scratch_types=[
            pltpu.VMEM((CHUNK,), jnp.float32),
            pltpu.VMEM((TILE + 16,), jnp.int32),
            pltpu.VMEM((TILE,), jnp.float32),
            pltpu.VMEM((NW + 16,), jnp.int32),
            pltpu.VMEM((NW + 16,), jnp.int32),
            pltpu.SemaphoreType.DMA,
            pltpu.SemaphoreType.DMA,
        ],
    )
    def k(cand_hbm, s_hbm, bias_hbm, lo_hbm, hi_hbm, out_hbm,
          chunk, cbuf, sbuf, lov, hiv, csem, ssem):
        cid = lax.axis_index("c")
        sid = lax.axis_index("s")
        wid = cid * 16 + sid
        ostart = pl.multiple_of(wid * CHUNK, 8)
        oend = ostart + CHUNK
        estart = pl.multiple_of(lax.rem(ostart, E), 8)
        pltpu.sync_copy(bias_hbm.at[pl.ds(estart, CHUNK)], chunk)
        pltpu.sync_copy(lo_hbm, lov)
        pltpu.sync_copy(hi_hbm, hiv)
        lo = lov[pl.ds(wid, 16)][0]
        hi = hiv[pl.ds(wid, 16)][0]
        lo8 = pl.multiple_of(lo - lax.rem(lo, 8), 8)
        ntiles = (hi - lo8 + (TILE - 1)) // TILE

        def tile_body(t, carry):
            base = pl.multiple_of(lo8 + t * TILE, 8)
            cc = pltpu.async_copy(cand_hbm.at[pl.ds(base, TILE + 16)], cbuf, csem)
            sc = pltpu.async_copy(s_hbm.at[pl.ds(base, TILE)], sbuf, ssem)
            cc.wait()
            sc.wait()
            for g in range(GROUPS):
                off = g * 16
                vc = cbuf[pl.ds(off, 16)]
                vn = cbuf[pl.ds(off + 1, 16)]
                vs = sbuf[pl.ds(off, 16)]
                idx = vc - ostart
                keep = (vc >= ostart) & (vc < oend) & (vc != vn)
                bv = plsc.load_gather(chunk, [idx], mask=keep)
                plsc.store_scatter(chunk, [idx], vs + bv, mask=keep)
            return carry

        lax.fori_loop(0, ntiles, tile_body, 0)
        pltpu.sync_copy(chunk, out_hbm.at[pl.ds(ostart, CHUNK)])

    return k(cand_pad, s_pad, bias, lo_arr, hi_arr)


def kernel(rule_count, rule_emb, candidate_set, all_r, relation_table,
           ln_gamma, ln_beta, W1, b1, W2, b2, bias):
    rel = relation_table[all_r].reshape(H, 1)
    s2d = _dense_scores(rule_count, rule_emb, rel,
                        ln_gamma.reshape(H, 1), ln_beta.reshape(H, 1),
                        W1, b1.reshape(128, 1), W2, b2.reshape(1, 1))
    s_pad = s2d.reshape(CPAD)
    cand_pad = jnp.concatenate(
        [candidate_set, jnp.full((CPAD - C,), BE, dtype=jnp.int32)])
    edges = jnp.arange(NW + 1, dtype=jnp.int32) * CHUNK
    bounds = jnp.searchsorted(candidate_set, edges, side="left",
                              method="compare_all").astype(jnp.int32)
    zpad = jnp.zeros((15,), jnp.int32)
    lo_arr = jnp.concatenate([bounds[:NW], zpad, zpad[:1]])
    hi_arr = jnp.concatenate([bounds[1:], zpad, zpad[:1]])
    out_flat = _sc_scatter(cand_pad, s_pad, bias, lo_arr, hi_arr)
    return out_flat.reshape(B, E)


# speedup vs baseline: 11.0901x; 1.0418x over previous
import functools

import jax
import jax.numpy as jnp
from jax import lax
from jax.experimental import pallas as pl
from jax.experimental.pallas import tpu as pltpu
from jax.experimental.pallas import tpu_sc as plsc

R, C, H, B, E = 64, 200000, 16, 16, 50000
BE = B * E
BLK = 16384
GRID = (C + BLK - 1) // BLK
CPAD = GRID * BLK
NW = 32
CHUNK = BE // NW
TILE = 2048
GROUPS = TILE // 16


def _dense_body(rc_ref, g_ref, gam_ref, bet_ref, w1a_ref, hrel_ref,
                w2_ref, b2_ref, s_ref):
    rc = rc_ref[...]
    mg = lax.dot_general(g_ref[...], rc, (((0,), (0,)), ((), ())),
                         preferred_element_type=jnp.float32)
    msg = mg[:H]
    r = 1.0 / (mg[H:H + 1] + 1e-6)
    x = msg * r
    mu = jnp.mean(x, axis=0, keepdims=True)
    var = jnp.mean((x - mu) * (x - mu), axis=0, keepdims=True)
    x = (x - mu) * lax.rsqrt(var + 1e-5) * gam_ref[...] + bet_ref[...]
    x = jnp.maximum(x, 0.0)
    h = lax.dot_general(w1a_ref[...], x, (((0,), (0,)), ((), ())),
                        preferred_element_type=jnp.float32)
    h = jnp.maximum(h + hrel_ref[...], 0.0)
    s = lax.dot_general(w2_ref[...], h, (((0,), (0,)), ((), ())),
                        preferred_element_type=jnp.float32)
    s_ref[...] = s + b2_ref[...]


def _dense_scores(rule_count, G, ln_gamma, ln_beta, W1a, h_rel, W2, b2):
    return pl.pallas_call(
        _dense_body,
        grid=(GRID,),
        in_specs=[
            pl.BlockSpec((R, BLK), lambda i: (0, i)),
            pl.BlockSpec((R, H + 1), lambda i: (0, 0)),
            pl.BlockSpec((H, 1), lambda i: (0, 0)),
            pl.BlockSpec((H, 1), lambda i: (0, 0)),
            pl.BlockSpec((H, 128), lambda i: (0, 0)),
            pl.BlockSpec((128, 1), lambda i: (0, 0)),
            pl.BlockSpec((128, 1), lambda i: (0, 0)),
            pl.BlockSpec((1, 1), lambda i: (0, 0)),
        ],
        out_specs=pl.BlockSpec((1, BLK), lambda i: (0, i)),
        out_shape=jax.ShapeDtypeStruct((1, CPAD), jnp.float32),
        compiler_params=pltpu.CompilerParams(
            dimension_semantics=("arbitrary",)),
    )(rule_count, G, ln_gamma, ln_beta, W1a, h_rel, W2, b2)


def _sc_scatter(cand_pad, s_pad, bias, lo_arr, hi_arr):
    mesh = plsc.VectorSubcoreMesh(core_axis_name="c", subcore_axis_name="s")

    @functools.partial(
        pl.kernel,
        mesh=mesh,
        out_type=jax.ShapeDtypeStruct((BE,), jnp.float32),
        compiler_params=pltpu.CompilerParams(needs_layout_passes=False),
        scratch_types=[
            pltpu.VMEM((CHUNK,), jnp.float32),
            pltpu.VMEM((TILE + 16,), jnp.int32),
            pltpu.VMEM((TILE,), jnp.float32),
            pltpu.VMEM((NW + 16,), jnp.int32),
            pltpu.VMEM((NW + 16,), jnp.int32),
            pltpu.SemaphoreType.DMA,
            pltpu.SemaphoreType.DMA,
            pltpu.SemaphoreType.DMA,
        ],
    )
    def k(cand_hbm, s_hbm, bias_hbm, lo_hbm, hi_hbm, out_hbm,
          chunk, cbuf, sbuf, lov, hiv, csem, ssem, bsem):
        cid = lax.axis_index("c")
        sid = lax.axis_index("s")
        wid = cid * 16 + sid
        ostart = pl.multiple_of(wid * CHUNK, 8)
        oend = ostart + CHUNK
        estart = pl.multiple_of(lax.rem(ostart, E), 8)
        bias_cp = pltpu.async_copy(bias_hbm.at[pl.ds(estart, CHUNK)], chunk,
                                   bsem)
        pltpu.sync_copy(lo_hbm, lov)
        pltpu.sync_copy(hi_hbm, hiv)
        lo = lov[pl.ds(wid, 16)][0]
        hi = hiv[pl.ds(wid, 16)][0]
        lo8 = pl.multiple_of(lo - lax.rem(lo, 8), 8)
        ntiles = (hi - lo8 + (TILE - 1)) // TILE
        bias_cp.wait()

        def tile_body(t, carry):
            base = pl.multiple_of(lo8 + t * TILE, 8)
            cc = pltpu.async_copy(cand_hbm.at[pl.ds(base, TILE + 16)], cbuf, csem)
            sc = pltpu.async_copy(s_hbm.at[pl.ds(base, TILE)], sbuf, ssem)
            cc.wait()
            sc.wait()
            for g in range(GROUPS):
                off = g * 16
                vc = cbuf[pl.ds(off, 16)]
                vn = cbuf[pl.ds(off + 1, 16)]
                vs = sbuf[pl.ds(off, 16)]
                idx = vc - ostart
                keep = (vc >= ostart) & (vc < oend) & (vc != vn)
                bv = plsc.load_gather(chunk, [idx], mask=keep)
                plsc.store_scatter(chunk, [idx], vs + bv, mask=keep)
            return carry

        lax.fori_loop(0, ntiles, tile_body, 0)
        pltpu.sync_copy(chunk, out_hbm.at[pl.ds(ostart, CHUNK)])

    return k(cand_pad, s_pad, bias, lo_arr, hi_arr)


def kernel(rule_count, rule_emb, candidate_set, all_r, relation_table,
           ln_gamma, ln_beta, W1, b1, W2, b2, bias):
    rel = relation_table[all_r]
    G = jnp.concatenate([rule_emb, jnp.ones((R, 1), jnp.float32)], axis=1)
    h_rel = (rel @ W1[H:]) + b1
    s2d = _dense_scores(rule_count, G,
                        ln_gamma.reshape(H, 1), ln_beta.reshape(H, 1),
                        W1[:H], h_rel.reshape(128, 1),
                        W2, b2.reshape(1, 1))
    s_pad = s2d.reshape(CPAD)
    cand_pad = jnp.concatenate(
        [candidate_set, jnp.full((CPAD - C,), BE, dtype=jnp.int32)])
    edges = jnp.arange(NW + 1, dtype=jnp.int32) * CHUNK
    bounds = jnp.searchsorted(candidate_set, edges, side="left",
                              method="compare_all").astype(jnp.int32)
    zpad = jnp.zeros((15,), jnp.int32)
    lo_arr = jnp.concatenate([bounds[:NW], zpad, zpad[:1]])
    hi_arr = jnp.concatenate([bounds[1:], zpad, zpad[:1]])
    out_flat = _sc_scatter(cand_pad, s_pad, bias, lo_arr, hi_arr)
    return out_flat.reshape(B, E)
